# baseline (speedup 1.0000x reference)
import functools

import jax
import jax.numpy as jnp
from jax import lax
from jax.experimental import pallas as pl
from jax.experimental.pallas import tpu as pltpu
from jax.experimental.pallas import tpu_sc as plsc

_NC = 2
_NS = 16
_NT = _NC * _NS
_L = 16
_CHUNK = 128
_D = 128


def _mesh():
    return plsc.VectorSubcoreMesh(core_axis_name="c", subcore_axis_name="s")


def _zero_vmem(ref, rows, width):
    def body(r, _):
        for k in range(width // _L):
            ref[r, pl.ds(k * _L, _L)] = jnp.zeros((_L,), jnp.float32)
        return 0
    lax.fori_loop(0, rows, body, 0)


def _sc_count(idx2, s_pad):
    n_chunks_total = idx2.shape[0]
    n_chunks = n_chunks_total // _NT
    s_rows = s_pad // _NS
    zrows = 40

    @functools.partial(
        pl.kernel,
        out_type=jax.ShapeDtypeStruct((_NC, s_pad, _D), jnp.float32),
        mesh=_mesh(),
        scratch_types=[
            pltpu.VMEM((_CHUNK,), jnp.int32),
            pltpu.VMEM((_CHUNK, _D), jnp.float32),
            pltpu.VMEM((zrows, _D), jnp.float32),
            pltpu.VMEM_SHARED((s_pad, _D), jnp.float32),
        ],
    )
    def k(i_hbm, o_hbm, idx, ones, zbuf, acc):
        cid = lax.axis_index("c")
        sid = lax.axis_index("s")
        wid = sid * _NC + cid

        def fill_ones(r, _):
            for kk in range(_D // _L):
                ones[r, pl.ds(kk * _L, _L)] = jnp.ones((_L,), jnp.float32)
            return 0
        lax.fori_loop(0, _CHUNK, fill_ones, 0)
        _zero_vmem(zbuf, zrows, _D)
        base = sid * s_rows
        for t in range(s_rows // zrows):
            pltpu.sync_copy(zbuf, acc.at[pl.ds(base + t * zrows, zrows)])
        cbase = wid * n_chunks
        plsc.subcore_barrier()

        def body(j, _):
            pltpu.sync_copy(i_hbm.at[cbase + j], idx)
            pltpu.sync_copy(ones, acc.at[idx], add=True)
            return 0
        lax.fori_loop(0, n_chunks, body, 0)

        plsc.subcore_barrier()
        pltpu.sync_copy(acc.at[pl.ds(base, s_rows)],
                        o_hbm.at[cid, pl.ds(base, s_rows)])

    return k(idx2)


def _tc_rcp(parts):
    s_pad = parts.shape[1]

    def body(a_ref, b_ref, o_ref):
        o_ref[...] = 1.0 / jnp.maximum(a_ref[...] + b_ref[...], 1.0)

    out = pl.pallas_call(
        body,
        out_shape=jax.ShapeDtypeStruct((s_pad, _D), jnp.float32),
    )(parts[0], parts[1])
    return out[:, :1]


def _tc_combine(parts, rcp_col, mode, mol=None):
    s_pad = parts.shape[1]
    block = 1280
    grid = s_pad // block
    row_spec = pl.BlockSpec((block, _D), lambda i: (i, 0))
    col_spec = pl.BlockSpec((block, 1), lambda i: (i, 0))

    def body(*refs):
        if mode == 2:
            a_ref, b_ref, r_ref, m_ref, o_ref = refs
        else:
            a_ref, b_ref, r_ref, o_ref = refs
        y = (a_ref[...] + b_ref[...]) * r_ref[...]
        if mode == 1:
            y = jnp.maximum(y, 0.0)
        if mode == 2:
            y = y + m_ref[...]
        o_ref[...] = y

    args = [parts[0], parts[1], rcp_col]
    in_specs = [row_spec, row_spec, col_spec]
    if mode == 2:
        args.append(mol)
        in_specs.append(row_spec)
    return pl.pallas_call(
        body,
        grid=(grid,),
        in_specs=in_specs,
        out_specs=row_spec,
        out_shape=jax.ShapeDtypeStruct((s_pad, _D), jnp.float32),
    )(*args)


def _sc_gather_rows(table, idx):
    n = idx.shape[0]
    rows_per_tile = n // _NT
    gchunk = 80
    n_chunks = rows_per_tile // gchunk

    @functools.partial(
        pl.kernel,
        out_type=jax.ShapeDtypeStruct((n, _D), jnp.float32),
        mesh=_mesh(),
        scratch_types=[
            pltpu.VMEM((gchunk,), jnp.int32),
            pltpu.VMEM((rows_per_tile, _D), jnp.float32),
            pltpu.SemaphoreType.DMA,
        ],
    )
    def k(t_hbm, i_hbm, o_hbm, gidx, rows, sem):
        cid = lax.axis_index("c")
        sid = lax.axis_index("s")
        wid = sid * _NC + cid
        base = wid * rows_per_tile

        def body(t, _):
            pltpu.sync_copy(i_hbm.at[pl.ds(base + t * gchunk, gchunk)], gidx)
            pltpu.async_copy(t_hbm.at[gidx], rows.at[pl.ds(t * gchunk, gchunk)],
                             sem).wait()
            return 0
        lax.fori_loop(0, n_chunks, body, 0)
        pltpu.sync_copy(rows, o_hbm.at[pl.ds(base, rows_per_tile)])

    return k(table, idx)


def _sc_segsum(x, src_idx2, dst_idx2, s_pad):
    n_chunks_total = src_idx2.shape[0]
    n_chunks = n_chunks_total // _NT
    rows_per_tile = s_pad // _NS

    @functools.partial(
        pl.kernel,
        out_type=jax.ShapeDtypeStruct((_NC, s_pad, _D), jnp.float32),
        mesh=_mesh(),
        scratch_types=[
            pltpu.VMEM((_CHUNK,), jnp.int32),
            pltpu.VMEM((_CHUNK,), jnp.int32),
            pltpu.VMEM((_CHUNK, _D), jnp.float32),
            pltpu.VMEM_SHARED((s_pad, _D), jnp.float32),
            pltpu.SemaphoreType.DMA,
        ],
    )
    def k(x_hbm, src_hbm, dst_hbm, out_hbm, sidx, didx, rows, acc, sem):
        cid = lax.axis_index("c")
        sid = lax.axis_index("s")
        wid = sid * _NC + cid
        _zero_vmem(rows, _CHUNK, _D)
        base = sid * rows_per_tile
        for t in range(rows_per_tile // _CHUNK):
            pltpu.sync_copy(rows, acc.at[pl.ds(base + t * _CHUNK, _CHUNK)])
        rem = rows_per_tile % _CHUNK
        if rem:
            pltpu.sync_copy(
                rows.at[pl.ds(0, rem)],
                acc.at[pl.ds(base + rows_per_tile - rem, rem)])
        cbase = wid * n_chunks
        plsc.subcore_barrier()

        def body(j, _):
            pltpu.sync_copy(src_hbm.at[cbase + j], sidx)
            pltpu.sync_copy(dst_hbm.at[cbase + j], didx)
            pltpu.async_copy(x_hbm.at[sidx], rows, sem).wait()
            pltpu.sync_copy(rows, acc.at[didx], add=True)
            return 0
        lax.fori_loop(0, n_chunks, body, 0)

        plsc.subcore_barrier()
        pltpu.sync_copy(acc.at[pl.ds(base, rows_per_tile)],
                        out_hbm.at[cid, pl.ds(base, rows_per_tile)])

    return k(x, src_idx2, dst_idx2)


def _tc_linear_relu(x, w, b):
    n, din = x.shape
    dout = w.shape[1]
    block = 1280
    grid = n // block

    def body(x_ref, w_ref, b_ref, o_ref):
        y = jnp.dot(x_ref[...], w_ref[...],
                    preferred_element_type=jnp.float32,
                    precision=lax.Precision.HIGHEST)
        o_ref[...] = jnp.maximum(y + b_ref[...], 0.0)

    return pl.pallas_call(
        body,
        grid=(grid,),
        in_specs=[
            pl.BlockSpec((block, din), lambda i: (i, 0)),
            pl.BlockSpec((din, dout), lambda i: (0, 0)),
            pl.BlockSpec((1, dout), lambda i: (0, 0)),
        ],
        out_specs=pl.BlockSpec((block, dout), lambda i: (i, 0)),
        out_shape=jax.ShapeDtypeStruct((n, dout), jnp.float32),
    )(x, w, b[None, :])


def _tc_tail(ft, yb, Wmu, bmu, Wlv, blv, W3, b3, gamma, beta, Wc, bc, Wd, bd):
    n = ft.shape[0]
    nl = yb.shape[1]
    zdim = Wmu.shape[1]
    block = 1000
    grid = n // block
    dotf = functools.partial(jnp.dot, preferred_element_type=jnp.float32,
                             precision=lax.Precision.HIGHEST)
    g2 = (gamma / jnp.sqrt(1.0 + 1e-5))[None, :]
    Wmu_x, Wmu_y = Wmu[:_D], Wmu[_D:]
    Wlv_x, Wlv_y = Wlv[:_D], Wlv[_D:]
    Wd_z, Wd_y = Wd[:zdim], Wd[zdim:]

    def body(ft_ref, yb_ref, wmux, wmuy, bmu_r, wlvx, wlvy, blv_r, w3, b3_r,
             g2_r, beta_r, wc, bc_r, wdz, wdy, bd_r,
             mu_o, lv_o, lg_o, xr_o):
        ftb = ft_ref[...]
        ybb = yb_ref[...]
        mu = dotf(ftb, wmux[...]) + dotf(ybb, wmuy[...]) + bmu_r[...]
        lv = dotf(ftb, wlvx[...]) + dotf(ybb, wlvy[...]) + blv_r[...]
        h = dotf(mu, w3[...]) + b3_r[...]
        h = jnp.maximum(g2_r[...] * h + beta_r[...], 0.0)
        lg = dotf(h, wc[...]) + bc_r[...]
        xr = dotf(mu, wdz[...]) + dotf(ybb, wdy[...]) + bd_r[...]
        mu_o[...] = mu
        lv_o[...] = lv
        lg_o[...] = lg
        xr_o[...] = xr

    row_spec = lambda d: pl.BlockSpec((block, d), lambda i: (i, 0))
    full = lambda a: pl.BlockSpec(a.shape, lambda i: (0,) * a.ndim)
    args = (ft, yb, Wmu_x, Wmu_y, bmu[None, :], Wlv_x, Wlv_y, blv[None, :],
            W3, b3[None, :], g2, beta[None, :], Wc, bc[None, :],
            Wd_z, Wd_y, bd[None, :])
    in_specs = [row_spec(_D), row_spec(nl)] + [full(a) for a in args[2:]]
    return pl.pallas_call(
        body,
        grid=(grid,),
        in_specs=in_specs,
        out_specs=(row_spec(zdim), row_spec(zdim), row_spec(nl), row_spec(_D)),
        out_shape=(
            jax.ShapeDtypeStruct((n, zdim), jnp.float32),
            jax.ShapeDtypeStruct((n, zdim), jnp.float32),
            jax.ShapeDtypeStruct((n, nl), jnp.float32),
            jax.ShapeDtypeStruct((n, _D), jnp.float32),
        ),
    )(*args)


def kernel(feature, v_idx, e_idx, y_bin, y_target, drug_matrix, new_data_idx,
           W1, b1, W2, b2, Wmu, bmu, Wlv, blv, W3, b3, gamma, beta, Wc, bc,
           Wd, bd):
    n_nodes, d = feature.shape
    nnz = v_idx.shape[0]
    n_he = 5000
    train = n_nodes - y_bin.shape[0]

    unit = _NT * _CHUNK * 2
    nnz_pad = ((nnz + unit - 1) // unit) * unit
    e_pad = ((n_he + 1 + 255) // 256) * 256
    v_pad = ((n_nodes + 1 + 255) // 256) * 256

    v_idx = v_idx.astype(jnp.int32)
    e_idx = e_idx.astype(jnp.int32)
    v_idx_p = jnp.concatenate(
        [v_idx, jnp.full((nnz_pad - nnz,), n_nodes, jnp.int32)])
    e_idx_p = jnp.concatenate(
        [e_idx, jnp.full((nnz_pad - nnz,), n_he, jnp.int32)])
    nidx_p = jnp.concatenate(
        [new_data_idx.astype(jnp.int32),
         jnp.zeros((v_pad - n_nodes,), jnp.int32)])
    feat_in = jnp.pad(feature, ((0, v_pad - n_nodes), (0, 0)))

    v_idx2 = v_idx_p.reshape(-1, _CHUNK)
    e_idx2 = e_idx_p.reshape(-1, _CHUNK)

    e_rcp = _tc_rcp(_sc_count(e_idx2, e_pad))
    v_rcp = _tc_rcp(_sc_count(v_idx2, v_pad))
    mol = _sc_gather_rows(drug_matrix, nidx_p)

    x1 = _tc_linear_relu(feat_in, W1, b1)
    ep = _sc_segsum(x1, v_idx2, e_idx2, e_pad)
    e_feat = _tc_combine(ep, e_rcp, mode=0)
    vp = _sc_segsum(e_feat, e_idx2, v_idx2, v_pad)
    x1m = _tc_combine(vp, v_rcp, mode=1)

    x2 = _tc_linear_relu(x1m, W2, b2)
    ep2 = _sc_segsum(x2, v_idx2, e_idx2, e_pad)
    e_feat2 = _tc_combine(ep2, e_rcp, mode=0)
    vp2 = _sc_segsum(e_feat2, e_idx2, v_idx2, v_pad)
    feat_full = _tc_combine(vp2, v_rcp, mode=2, mol=mol)

    feat = feat_full[:n_nodes]
    ft = feat_full[train:n_nodes]
    mu, logvar, logits, x_reconst = _tc_tail(
        ft, y_bin, Wmu, bmu, Wlv, blv, W3, b3, gamma, beta, Wc, bc, Wd, bd)
    return (mu, logvar, mu, logits, x_reconst, y_target, feat)


# speedup vs baseline: 1.3073x; 1.3073x over previous
import functools

import jax
import jax.numpy as jnp
from jax import lax
from jax.experimental import pallas as pl
from jax.experimental.pallas import tpu as pltpu
from jax.experimental.pallas import tpu_sc as plsc

_NC = 2
_NS = 16
_NT = _NC * _NS
_L = 16
_CHUNK = 128
_D = 128


def _mesh():
    return plsc.VectorSubcoreMesh(core_axis_name="c", subcore_axis_name="s")


def _zero_vmem(ref, rows, width):
    def body(r, _):
        for k in range(width // _L):
            ref[r, pl.ds(k * _L, _L)] = jnp.zeros((_L,), jnp.float32)
        return 0
    lax.fori_loop(0, rows, body, 0)


def _sc_count(idx_arr, s_pad):
    nnz_pad = idx_arr.shape[0]
    per_tile = nnz_pad // _NT
    n_chunks = per_tile // _CHUNK
    s_rows = s_pad // _NS
    zrows = 160

    @functools.partial(
        pl.kernel,
        out_type=jax.ShapeDtypeStruct((_NC, s_pad, _D), jnp.float32),
        mesh=_mesh(),
        scratch_types=[
            pltpu.VMEM((_CHUNK,), jnp.int32),
            pltpu.VMEM((_CHUNK, _D), jnp.float32),
            pltpu.VMEM((zrows, _D), jnp.float32),
            pltpu.VMEM_SHARED((s_pad, _D), jnp.float32),
        ],
    )
    def k(i_hbm, o_hbm, idx, ones, zbuf, acc):
        cid = lax.axis_index("c")
        sid = lax.axis_index("s")
        wid = sid * _NC + cid

        def fill_ones(r, _):
            for kk in range(_D // _L):
                ones[r, pl.ds(kk * _L, _L)] = jnp.ones((_L,), jnp.float32)
            return 0
        lax.fori_loop(0, _CHUNK, fill_ones, 0)
        _zero_vmem(zbuf, zrows, _D)
        base = sid * s_rows
        for t in range(s_rows // zrows):
            pltpu.sync_copy(zbuf, acc.at[pl.ds(base + t * zrows, zrows)])
        plsc.subcore_barrier()

        nbase = wid * per_tile

        def body(j, _):
            pltpu.sync_copy(i_hbm.at[pl.ds(nbase + j * _CHUNK, _CHUNK)], idx)
            pltpu.sync_copy(ones, acc.at[idx], add=True)
            return 0
        lax.fori_loop(0, n_chunks, body, 0)

        plsc.subcore_barrier()
        pltpu.sync_copy(acc.at[pl.ds(base, s_rows)],
                        o_hbm.at[cid, pl.ds(base, s_rows)])

    return k(idx_arr)


def _tc_rcp(parts):
    s_pad = parts.shape[1]

    def body(a_ref, b_ref, o_ref):
        o_ref[...] = 1.0 / jnp.maximum(a_ref[...] + b_ref[...], 1.0)

    out = pl.pallas_call(
        body,
        out_shape=jax.ShapeDtypeStruct((s_pad, _D), jnp.float32),
    )(parts[0], parts[1])
    return out[:, :1]


def _tc_combine(parts, rcp_col, mode, mol=None):
    s_pad = parts.shape[1]
    block = 1280
    grid = s_pad // block
    row_spec = pl.BlockSpec((block, _D), lambda i: (i, 0))
    col_spec = pl.BlockSpec((block, 1), lambda i: (i, 0))

    def body(*refs):
        if mode == 2:
            a_ref, b_ref, r_ref, m_ref, o_ref = refs
        else:
            a_ref, b_ref, r_ref, o_ref = refs
        y = (a_ref[...] + b_ref[...]) * r_ref[...]
        if mode == 1:
            y = jnp.maximum(y, 0.0)
        if mode == 2:
            y = y + m_ref[...]
        o_ref[...] = y

    args = [parts[0], parts[1], rcp_col]
    in_specs = [row_spec, row_spec, col_spec]
    if mode == 2:
        args.append(mol)
        in_specs.append(row_spec)
    return pl.pallas_call(
        body,
        grid=(grid,),
        in_specs=in_specs,
        out_specs=row_spec,
        out_shape=jax.ShapeDtypeStruct((s_pad, _D), jnp.float32),
    )(*args)


def _sc_gather_rows(table, idx):
    n = idx.shape[0]
    rows_per_tile = n // _NT
    gchunk = 80
    n_chunks = rows_per_tile // gchunk

    @functools.partial(
        pl.kernel,
        out_type=jax.ShapeDtypeStruct((n, _D), jnp.float32),
        mesh=_mesh(),
        scratch_types=[
            pltpu.VMEM((gchunk,), jnp.int32),
            pltpu.VMEM((rows_per_tile, _D), jnp.float32),
            pltpu.SemaphoreType.DMA,
        ],
    )
    def k(t_hbm, i_hbm, o_hbm, gidx, rows, sem):
        cid = lax.axis_index("c")
        sid = lax.axis_index("s")
        wid = sid * _NC + cid
        base = wid * rows_per_tile

        def body(t, _):
            pltpu.sync_copy(i_hbm.at[pl.ds(base + t * gchunk, gchunk)], gidx)
            pltpu.async_copy(t_hbm.at[gidx], rows.at[pl.ds(t * gchunk, gchunk)],
                             sem).wait()
            return 0
        lax.fori_loop(0, n_chunks, body, 0)
        pltpu.sync_copy(rows, o_hbm.at[pl.ds(base, rows_per_tile)])

    return k(table, idx)


def _sc_segsum(x, src_idx, dst_idx, s_pad):
    nnz_pad = src_idx.shape[0]
    per_tile = nnz_pad // _NT
    n_chunks = per_tile // _CHUNK
    rows_per_tile = s_pad // _NS
    zrows = 160

    @functools.partial(
        pl.kernel,
        out_type=jax.ShapeDtypeStruct((_NC, s_pad, _D), jnp.float32),
        mesh=_mesh(),
        scratch_types=[
            pltpu.VMEM((_CHUNK,), jnp.int32),
            pltpu.VMEM((_CHUNK,), jnp.int32),
            pltpu.VMEM((_CHUNK, _D), jnp.float32),
            pltpu.VMEM((zrows, _D), jnp.float32),
            pltpu.VMEM_SHARED((s_pad, _D), jnp.float32),
            pltpu.SemaphoreType.DMA,
        ],
    )
    def k(x_hbm, src_hbm, dst_hbm, out_hbm, sidx, didx, rows, zbuf, acc, sem):
        cid = lax.axis_index("c")
        sid = lax.axis_index("s")
        wid = sid * _NC + cid
        _zero_vmem(zbuf, zrows, _D)
        base = sid * rows_per_tile
        for t in range(rows_per_tile // zrows):
            pltpu.sync_copy(zbuf, acc.at[pl.ds(base + t * zrows, zrows)])
        plsc.subcore_barrier()

        nbase = wid * per_tile

        def body(j, _):
            off = nbase + j * _CHUNK
            pltpu.sync_copy(src_hbm.at[pl.ds(off, _CHUNK)], sidx)
            pltpu.sync_copy(dst_hbm.at[pl.ds(off, _CHUNK)], didx)
            pltpu.async_copy(x_hbm.at[sidx], rows, sem).wait()
            pltpu.sync_copy(rows, acc.at[didx], add=True)
            return 0
        lax.fori_loop(0, n_chunks, body, 0)

        plsc.subcore_barrier()
        pltpu.sync_copy(acc.at[pl.ds(base, rows_per_tile)],
                        out_hbm.at[cid, pl.ds(base, rows_per_tile)])

    return k(x, src_idx, dst_idx)


def _tc_linear_relu(x, w, b):
    n, din = x.shape
    dout = w.shape[1]
    block = 1280
    grid = n // block

    def body(x_ref, w_ref, b_ref, o_ref):
        y = jnp.dot(x_ref[...], w_ref[...],
                    preferred_element_type=jnp.float32,
                    precision=lax.Precision.HIGHEST)
        o_ref[...] = jnp.maximum(y + b_ref[...], 0.0)

    return pl.pallas_call(
        body,
        grid=(grid,),
        in_specs=[
            pl.BlockSpec((block, din), lambda i: (i, 0)),
            pl.BlockSpec((din, dout), lambda i: (0, 0)),
            pl.BlockSpec((1, dout), lambda i: (0, 0)),
        ],
        out_specs=pl.BlockSpec((block, dout), lambda i: (i, 0)),
        out_shape=jax.ShapeDtypeStruct((n, dout), jnp.float32),
    )(x, w, b[None, :])


def _tc_tail(ft, yb, Wmu, bmu, Wlv, blv, W3, b3, gamma, beta, Wc, bc, Wd, bd):
    n = ft.shape[0]
    nl = yb.shape[1]
    zdim = Wmu.shape[1]
    block = 1000
    grid = n // block
    dotf = functools.partial(jnp.dot, preferred_element_type=jnp.float32,
                             precision=lax.Precision.HIGHEST)
    g2 = (gamma / jnp.sqrt(1.0 + 1e-5))[None, :]
    Wmu_x, Wmu_y = Wmu[:_D], Wmu[_D:]
    Wlv_x, Wlv_y = Wlv[:_D], Wlv[_D:]
    Wd_z, Wd_y = Wd[:zdim], Wd[zdim:]

    def body(ft_ref, yb_ref, wmux, wmuy, bmu_r, wlvx, wlvy, blv_r, w3, b3_r,
             g2_r, beta_r, wc, bc_r, wdz, wdy, bd_r,
             mu_o, lv_o, lg_o, xr_o):
        ftb = ft_ref[...]
        ybb = yb_ref[...]
        mu = dotf(ftb, wmux[...]) + dotf(ybb, wmuy[...]) + bmu_r[...]
        lv = dotf(ftb, wlvx[...]) + dotf(ybb, wlvy[...]) + blv_r[...]
        h = dotf(mu, w3[...]) + b3_r[...]
        h = jnp.maximum(g2_r[...] * h + beta_r[...], 0.0)
        lg = dotf(h, wc[...]) + bc_r[...]
        xr = dotf(mu, wdz[...]) + dotf(ybb, wdy[...]) + bd_r[...]
        mu_o[...] = mu
        lv_o[...] = lv
        lg_o[...] = lg
        xr_o[...] = xr

    row_spec = lambda d: pl.BlockSpec((block, d), lambda i: (i, 0))
    full = lambda a: pl.BlockSpec(a.shape, lambda i: (0,) * a.ndim)
    args = (ft, yb, Wmu_x, Wmu_y, bmu[None, :], Wlv_x, Wlv_y, blv[None, :],
            W3, b3[None, :], g2, beta[None, :], Wc, bc[None, :],
            Wd_z, Wd_y, bd[None, :])
    in_specs = [row_spec(_D), row_spec(nl)] + [full(a) for a in args[2:]]
    return pl.pallas_call(
        body,
        grid=(grid,),
        in_specs=in_specs,
        out_specs=(row_spec(zdim), row_spec(zdim), row_spec(nl), row_spec(_D)),
        out_shape=(
            jax.ShapeDtypeStruct((n, zdim), jnp.float32),
            jax.ShapeDtypeStruct((n, zdim), jnp.float32),
            jax.ShapeDtypeStruct((n, nl), jnp.float32),
            jax.ShapeDtypeStruct((n, _D), jnp.float32),
        ),
    )(*args)


def kernel(feature, v_idx, e_idx, y_bin, y_target, drug_matrix, new_data_idx,
           W1, b1, W2, b2, Wmu, bmu, Wlv, blv, W3, b3, gamma, beta, Wc, bc,
           Wd, bd):
    n_nodes, d = feature.shape
    nnz = v_idx.shape[0]
    n_he = 5000
    train = n_nodes - y_bin.shape[0]

    unit = _NT * _CHUNK
    nnz_pad = ((nnz + unit - 1) // unit) * unit
    e_pad = ((n_he + 1 + 255) // 256) * 256
    v_pad = ((n_nodes + 1 + 255) // 256) * 256

    v_idx = v_idx.astype(jnp.int32)
    e_idx = e_idx.astype(jnp.int32)
    v_idx_p = jnp.concatenate(
        [v_idx, jnp.full((nnz_pad - nnz,), n_nodes, jnp.int32)])
    e_idx_p = jnp.concatenate(
        [e_idx, jnp.full((nnz_pad - nnz,), n_he, jnp.int32)])
    nidx_p = jnp.concatenate(
        [new_data_idx.astype(jnp.int32),
         jnp.zeros((v_pad - n_nodes,), jnp.int32)])
    feat_in = jnp.pad(feature, ((0, v_pad - n_nodes), (0, 0)))

    e_rcp = _tc_rcp(_sc_count(e_idx_p, e_pad))
    v_rcp = _tc_rcp(_sc_count(v_idx_p, v_pad))
    mol = _sc_gather_rows(drug_matrix, nidx_p)

    x1 = _tc_linear_relu(feat_in, W1, b1)
    ep = _sc_segsum(x1, v_idx_p, e_idx_p, e_pad)
    e_feat = _tc_combine(ep, e_rcp, mode=0)
    vp = _sc_segsum(e_feat, e_idx_p, v_idx_p, v_pad)
    x1m = _tc_combine(vp, v_rcp, mode=1)

    x2 = _tc_linear_relu(x1m, W2, b2)
    ep2 = _sc_segsum(x2, v_idx_p, e_idx_p, e_pad)
    e_feat2 = _tc_combine(ep2, e_rcp, mode=0)
    vp2 = _sc_segsum(e_feat2, e_idx_p, v_idx_p, v_pad)
    feat_full = _tc_combine(vp2, v_rcp, mode=2, mol=mol)

    feat = feat_full[:n_nodes]
    ft = feat_full[train:n_nodes]
    mu, logvar, logits, x_reconst = _tc_tail(
        ft, y_bin, Wmu, bmu, Wlv, blv, W3, b3, gamma, beta, Wc, bc, Wd, bd)
    return (mu, logvar, mu, logits, x_reconst, y_target, feat)


# speedup vs baseline: 1.3938x; 1.0661x over previous
import functools

import jax
import jax.numpy as jnp
from jax import lax
from jax.experimental import pallas as pl
from jax.experimental.pallas import tpu as pltpu
from jax.experimental.pallas import tpu_sc as plsc

_NC = 2
_NS = 16
_NT = _NC * _NS
_L = 16
_CHUNK = 128
_D = 128


def _mesh():
    return plsc.VectorSubcoreMesh(core_axis_name="c", subcore_axis_name="s")


def _zero_vmem(ref, rows, width):
    def body(r, _):
        for k in range(width // _L):
            ref[r, pl.ds(k * _L, _L)] = jnp.zeros((_L,), jnp.float32)
        return 0
    lax.fori_loop(0, rows, body, 0)


def _sc_count(idx_arr, s_pad):
    nnz_pad = idx_arr.shape[0]
    per_tile = nnz_pad // _NT
    n_chunks = per_tile // _CHUNK
    s_rows = s_pad // _NS
    zrows = 160

    @functools.partial(
        pl.kernel,
        out_type=jax.ShapeDtypeStruct((_NC, s_pad, _D), jnp.float32),
        mesh=_mesh(),
        scratch_types=[
            pltpu.VMEM((_CHUNK,), jnp.int32),
            pltpu.VMEM((_CHUNK, _D), jnp.float32),
            pltpu.VMEM((zrows, _D), jnp.float32),
            pltpu.VMEM_SHARED((s_pad, _D), jnp.float32),
        ],
    )
    def k(i_hbm, o_hbm, idx, ones, zbuf, acc):
        cid = lax.axis_index("c")
        sid = lax.axis_index("s")
        wid = sid * _NC + cid

        def fill_ones(r, _):
            for kk in range(_D // _L):
                ones[r, pl.ds(kk * _L, _L)] = jnp.ones((_L,), jnp.float32)
            return 0
        lax.fori_loop(0, _CHUNK, fill_ones, 0)
        _zero_vmem(zbuf, zrows, _D)
        base = sid * s_rows
        for t in range(s_rows // zrows):
            pltpu.sync_copy(zbuf, acc.at[pl.ds(base + t * zrows, zrows)])
        plsc.subcore_barrier()

        nbase = wid * per_tile

        def body(j, _):
            pltpu.sync_copy(i_hbm.at[pl.ds(nbase + j * _CHUNK, _CHUNK)], idx)
            pltpu.sync_copy(ones, acc.at[idx], add=True)
            return 0
        lax.fori_loop(0, n_chunks, body, 0)

        plsc.subcore_barrier()
        pltpu.sync_copy(acc.at[pl.ds(base, s_rows)],
                        o_hbm.at[cid, pl.ds(base, s_rows)])

    return k(idx_arr)


def _tc_rcp(parts):
    s_pad = parts.shape[1]

    def body(a_ref, b_ref, o_ref):
        o_ref[...] = 1.0 / jnp.maximum(a_ref[...] + b_ref[...], 1.0)

    out = pl.pallas_call(
        body,
        out_shape=jax.ShapeDtypeStruct((s_pad, _D), jnp.float32),
    )(parts[0], parts[1])
    return out[:, :1]


def _tc_combine(parts, rcp_col, mode, mol=None):
    s_pad = parts.shape[1]
    block = 1280
    grid = s_pad // block
    row_spec = pl.BlockSpec((block, _D), lambda i: (i, 0))
    col_spec = pl.BlockSpec((block, 1), lambda i: (i, 0))

    def body(*refs):
        if mode == 2:
            a_ref, b_ref, r_ref, m_ref, o_ref = refs
        else:
            a_ref, b_ref, r_ref, o_ref = refs
        y = (a_ref[...] + b_ref[...]) * r_ref[...]
        if mode == 1:
            y = jnp.maximum(y, 0.0)
        if mode == 2:
            y = y + m_ref[...]
        o_ref[...] = y

    args = [parts[0], parts[1], rcp_col]
    in_specs = [row_spec, row_spec, col_spec]
    if mode == 2:
        args.append(mol)
        in_specs.append(row_spec)
    return pl.pallas_call(
        body,
        grid=(grid,),
        in_specs=in_specs,
        out_specs=row_spec,
        out_shape=jax.ShapeDtypeStruct((s_pad, _D), jnp.float32),
    )(*args)


def _sc_gather_rows(table, idx):
    n = idx.shape[0]
    rows_per_tile = n // _NT
    gchunk = 80
    n_chunks = rows_per_tile // gchunk

    @functools.partial(
        pl.kernel,
        out_type=jax.ShapeDtypeStruct((n, _D), jnp.float32),
        mesh=_mesh(),
        scratch_types=[
            pltpu.VMEM((gchunk,), jnp.int32),
            pltpu.VMEM((rows_per_tile, _D), jnp.float32),
            pltpu.SemaphoreType.DMA,
        ],
    )
    def k(t_hbm, i_hbm, o_hbm, gidx, rows, sem):
        cid = lax.axis_index("c")
        sid = lax.axis_index("s")
        wid = sid * _NC + cid
        base = wid * rows_per_tile

        def body(t, _):
            pltpu.sync_copy(i_hbm.at[pl.ds(base + t * gchunk, gchunk)], gidx)
            pltpu.async_copy(t_hbm.at[gidx], rows.at[pl.ds(t * gchunk, gchunk)],
                             sem).wait()
            return 0
        lax.fori_loop(0, n_chunks, body, 0)
        pltpu.sync_copy(rows, o_hbm.at[pl.ds(base, rows_per_tile)])

    return k(table, idx)


def _sc_segsum(x, src_idx, dst_idx, s_pad):
    nnz_pad = src_idx.shape[0]
    per_tile = nnz_pad // _NT
    n_chunks = per_tile // _CHUNK
    rows_per_tile = s_pad // _NS
    zrows = 160

    @functools.partial(
        pl.kernel,
        out_type=jax.ShapeDtypeStruct((_NC, s_pad, _D), jnp.float32),
        mesh=_mesh(),
        scratch_types=[
            pltpu.VMEM((_CHUNK,), jnp.int32),
            pltpu.VMEM((_CHUNK,), jnp.int32),
            pltpu.VMEM((_CHUNK, _D), jnp.float32),
            pltpu.VMEM((zrows, _D), jnp.float32),
            pltpu.VMEM_SHARED((s_pad, _D), jnp.float32),
            pltpu.SemaphoreType.DMA,
            pltpu.SemaphoreType.DMA,
        ],
    )
    def k(x_hbm, src_hbm, dst_hbm, out_hbm, sidx, didx, rows, zbuf, acc, sem,
          isem):
        cid = lax.axis_index("c")
        sid = lax.axis_index("s")
        wid = sid * _NC + cid
        _zero_vmem(zbuf, zrows, _D)
        base = sid * rows_per_tile
        for t in range(rows_per_tile // zrows):
            pltpu.sync_copy(zbuf, acc.at[pl.ds(base + t * zrows, zrows)])
        plsc.subcore_barrier()

        nbase = wid * per_tile

        def body(j, _):
            off = nbase + j * _CHUNK
            ca = pltpu.async_copy(src_hbm.at[pl.ds(off, _CHUNK)], sidx, isem)
            cb = pltpu.async_copy(dst_hbm.at[pl.ds(off, _CHUNK)], didx, isem)
            ca.wait()
            cb.wait()
            pltpu.async_copy(x_hbm.at[sidx], rows, sem).wait()
            pltpu.sync_copy(rows, acc.at[didx], add=True)
            return 0
        lax.fori_loop(0, n_chunks, body, 0)

        plsc.subcore_barrier()
        pltpu.sync_copy(acc.at[pl.ds(base, rows_per_tile)],
                        out_hbm.at[cid, pl.ds(base, rows_per_tile)])

    return k(x, src_idx, dst_idx)


def _tc_linear_relu(x, w, b):
    n, din = x.shape
    dout = w.shape[1]
    block = 1280
    grid = n // block

    def body(x_ref, w_ref, b_ref, o_ref):
        y = jnp.dot(x_ref[...], w_ref[...],
                    preferred_element_type=jnp.float32,
                    precision=lax.Precision.HIGHEST)
        o_ref[...] = jnp.maximum(y + b_ref[...], 0.0)

    return pl.pallas_call(
        body,
        grid=(grid,),
        in_specs=[
            pl.BlockSpec((block, din), lambda i: (i, 0)),
            pl.BlockSpec((din, dout), lambda i: (0, 0)),
            pl.BlockSpec((1, dout), lambda i: (0, 0)),
        ],
        out_specs=pl.BlockSpec((block, dout), lambda i: (i, 0)),
        out_shape=jax.ShapeDtypeStruct((n, dout), jnp.float32),
    )(x, w, b[None, :])


def _tc_tail(ft, yb, Wmu, bmu, Wlv, blv, W3, b3, gamma, beta, Wc, bc, Wd, bd):
    n = ft.shape[0]
    nl = yb.shape[1]
    zdim = Wmu.shape[1]
    block = 1000
    grid = n // block
    dotf = functools.partial(jnp.dot, preferred_element_type=jnp.float32,
                             precision=lax.Precision.HIGHEST)
    g2 = (gamma / jnp.sqrt(1.0 + 1e-5))[None, :]
    Wmu_x, Wmu_y = Wmu[:_D], Wmu[_D:]
    Wlv_x, Wlv_y = Wlv[:_D], Wlv[_D:]
    Wd_z, Wd_y = Wd[:zdim], Wd[zdim:]

    def body(ft_ref, yb_ref, wmux, wmuy, bmu_r, wlvx, wlvy, blv_r, w3, b3_r,
             g2_r, beta_r, wc, bc_r, wdz, wdy, bd_r,
             mu_o, lv_o, lg_o, xr_o):
        ftb = ft_ref[...]
        ybb = yb_ref[...]
        mu = dotf(ftb, wmux[...]) + dotf(ybb, wmuy[...]) + bmu_r[...]
        lv = dotf(ftb, wlvx[...]) + dotf(ybb, wlvy[...]) + blv_r[...]
        h = dotf(mu, w3[...]) + b3_r[...]
        h = jnp.maximum(g2_r[...] * h + beta_r[...], 0.0)
        lg = dotf(h, wc[...]) + bc_r[...]
        xr = dotf(mu, wdz[...]) + dotf(ybb, wdy[...]) + bd_r[...]
        mu_o[...] = mu
        lv_o[...] = lv
        lg_o[...] = lg
        xr_o[...] = xr

    row_spec = lambda d: pl.BlockSpec((block, d), lambda i: (i, 0))
    full = lambda a: pl.BlockSpec(a.shape, lambda i: (0,) * a.ndim)
    args = (ft, yb, Wmu_x, Wmu_y, bmu[None, :], Wlv_x, Wlv_y, blv[None, :],
            W3, b3[None, :], g2, beta[None, :], Wc, bc[None, :],
            Wd_z, Wd_y, bd[None, :])
    in_specs = [row_spec(_D), row_spec(nl)] + [full(a) for a in args[2:]]
    return pl.pallas_call(
        body,
        grid=(grid,),
        in_specs=in_specs,
        out_specs=(row_spec(zdim), row_spec(zdim), row_spec(nl), row_spec(_D)),
        out_shape=(
            jax.ShapeDtypeStruct((n, zdim), jnp.float32),
            jax.ShapeDtypeStruct((n, zdim), jnp.float32),
            jax.ShapeDtypeStruct((n, nl), jnp.float32),
            jax.ShapeDtypeStruct((n, _D), jnp.float32),
        ),
    )(*args)


def kernel(feature, v_idx, e_idx, y_bin, y_target, drug_matrix, new_data_idx,
           W1, b1, W2, b2, Wmu, bmu, Wlv, blv, W3, b3, gamma, beta, Wc, bc,
           Wd, bd):
    n_nodes, d = feature.shape
    nnz = v_idx.shape[0]
    n_he = 5000
    train = n_nodes - y_bin.shape[0]

    unit = _NT * _CHUNK
    nnz_pad = ((nnz + unit - 1) // unit) * unit
    e_pad = ((n_he + 1 + 255) // 256) * 256
    v_pad = ((n_nodes + 1 + 255) // 256) * 256

    v_idx = v_idx.astype(jnp.int32)
    e_idx = e_idx.astype(jnp.int32)
    v_idx_p = jnp.concatenate(
        [v_idx, jnp.full((nnz_pad - nnz,), n_nodes, jnp.int32)])
    e_idx_p = jnp.concatenate(
        [e_idx, jnp.full((nnz_pad - nnz,), n_he, jnp.int32)])
    nidx_p = jnp.concatenate(
        [new_data_idx.astype(jnp.int32),
         jnp.zeros((v_pad - n_nodes,), jnp.int32)])
    feat_in = jnp.pad(feature, ((0, v_pad - n_nodes), (0, 0)))

    e_rcp = _tc_rcp(_sc_count(e_idx_p, e_pad))
    v_rcp = _tc_rcp(_sc_count(v_idx_p, v_pad))
    mol = _sc_gather_rows(drug_matrix, nidx_p)

    x1 = _tc_linear_relu(feat_in, W1, b1)
    ep = _sc_segsum(x1, v_idx_p, e_idx_p, e_pad)
    e_feat = _tc_combine(ep, e_rcp, mode=0)
    vp = _sc_segsum(e_feat, e_idx_p, v_idx_p, v_pad)
    x1m = _tc_combine(vp, v_rcp, mode=1)

    x2 = _tc_linear_relu(x1m, W2, b2)
    ep2 = _sc_segsum(x2, v_idx_p, e_idx_p, e_pad)
    e_feat2 = _tc_combine(ep2, e_rcp, mode=0)
    vp2 = _sc_segsum(e_feat2, e_idx_p, v_idx_p, v_pad)
    feat_full = _tc_combine(vp2, v_rcp, mode=2, mol=mol)

    feat = feat_full[:n_nodes]
    ft = feat_full[train:n_nodes]
    mu, logvar, logits, x_reconst = _tc_tail(
        ft, y_bin, Wmu, bmu, Wlv, blv, W3, b3, gamma, beta, Wc, bc, Wd, bd)
    return (mu, logvar, mu, logits, x_reconst, y_target, feat)


# speedup vs baseline: 1.4919x; 1.0704x over previous
import functools

import jax
import jax.numpy as jnp
from jax import lax
from jax.experimental import pallas as pl
from jax.experimental.pallas import tpu as pltpu
from jax.experimental.pallas import tpu_sc as plsc

_NC = 2
_NS = 16
_NT = _NC * _NS
_L = 16
_CHUNK = 128
_D = 128


def _mesh():
    return plsc.VectorSubcoreMesh(core_axis_name="c", subcore_axis_name="s")


def _zero_vmem(ref, rows, width):
    def body(r, _):
        for k in range(width // _L):
            ref[r, pl.ds(k * _L, _L)] = jnp.zeros((_L,), jnp.float32)
        return 0
    lax.fori_loop(0, rows, body, 0)


def _sc_count(idx_arr, s_pad):
    nnz_pad = idx_arr.shape[0]
    per_tile = nnz_pad // _NT
    n_chunks = per_tile // _CHUNK
    s_rows = s_pad // _NS
    zrows = 160

    @functools.partial(
        pl.kernel,
        out_type=jax.ShapeDtypeStruct((_NC, s_pad, _D), jnp.float32),
        mesh=_mesh(),
        scratch_types=[
            pltpu.VMEM((_CHUNK,), jnp.int32),
            pltpu.VMEM((_CHUNK, _D), jnp.float32),
            pltpu.VMEM((zrows, _D), jnp.float32),
            pltpu.VMEM_SHARED((s_pad, _D), jnp.float32),
        ],
    )
    def k(i_hbm, o_hbm, idx, ones, zbuf, acc):
        cid = lax.axis_index("c")
        sid = lax.axis_index("s")
        wid = sid * _NC + cid

        def fill_ones(r, _):
            for kk in range(_D // _L):
                ones[r, pl.ds(kk * _L, _L)] = jnp.ones((_L,), jnp.float32)
            return 0
        lax.fori_loop(0, _CHUNK, fill_ones, 0)
        _zero_vmem(zbuf, zrows, _D)
        base = sid * s_rows
        for t in range(s_rows // zrows):
            pltpu.sync_copy(zbuf, acc.at[pl.ds(base + t * zrows, zrows)])
        plsc.subcore_barrier()

        nbase = wid * per_tile

        def body(j, _):
            pltpu.sync_copy(i_hbm.at[pl.ds(nbase + j * _CHUNK, _CHUNK)], idx)
            pltpu.sync_copy(ones, acc.at[idx], add=True)
            return 0
        lax.fori_loop(0, n_chunks, body, 0)

        plsc.subcore_barrier()
        pltpu.sync_copy(acc.at[pl.ds(base, s_rows)],
                        o_hbm.at[cid, pl.ds(base, s_rows)])

    return k(idx_arr)


def _tc_rcp(parts):
    s_pad = parts.shape[1]

    def body(a_ref, b_ref, o_ref):
        o_ref[...] = 1.0 / jnp.maximum(a_ref[...] + b_ref[...], 1.0)

    out = pl.pallas_call(
        body,
        out_shape=jax.ShapeDtypeStruct((s_pad, _D), jnp.float32),
    )(parts[0], parts[1])
    return out[:, :1]


def _tc_combine(parts, rcp_col, mode, mol=None):
    s_pad = parts.shape[1]
    block = 1280
    grid = s_pad // block
    row_spec = pl.BlockSpec((block, _D), lambda i: (i, 0))
    col_spec = pl.BlockSpec((block, 1), lambda i: (i, 0))

    def body(*refs):
        if mode == 2:
            a_ref, b_ref, r_ref, m_ref, o_ref = refs
        else:
            a_ref, b_ref, r_ref, o_ref = refs
        y = (a_ref[...] + b_ref[...]) * r_ref[...]
        if mode == 1:
            y = jnp.maximum(y, 0.0)
        if mode == 2:
            y = y + m_ref[...]
        o_ref[...] = y

    args = [parts[0], parts[1], rcp_col]
    in_specs = [row_spec, row_spec, col_spec]
    if mode == 2:
        args.append(mol)
        in_specs.append(row_spec)
    return pl.pallas_call(
        body,
        grid=(grid,),
        in_specs=in_specs,
        out_specs=row_spec,
        out_shape=jax.ShapeDtypeStruct((s_pad, _D), jnp.float32),
    )(*args)


def _sc_gather_rows(table, idx):
    n = idx.shape[0]
    rows_per_tile = n // _NT
    gchunk = 80
    n_chunks = rows_per_tile // gchunk

    @functools.partial(
        pl.kernel,
        out_type=jax.ShapeDtypeStruct((n, _D), jnp.float32),
        mesh=_mesh(),
        scratch_types=[
            pltpu.VMEM((gchunk,), jnp.int32),
            pltpu.VMEM((rows_per_tile, _D), jnp.float32),
            pltpu.SemaphoreType.DMA,
        ],
    )
    def k(t_hbm, i_hbm, o_hbm, gidx, rows, sem):
        cid = lax.axis_index("c")
        sid = lax.axis_index("s")
        wid = sid * _NC + cid
        base = wid * rows_per_tile

        def body(t, _):
            pltpu.sync_copy(i_hbm.at[pl.ds(base + t * gchunk, gchunk)], gidx)
            pltpu.async_copy(t_hbm.at[gidx], rows.at[pl.ds(t * gchunk, gchunk)],
                             sem).wait()
            return 0
        lax.fori_loop(0, n_chunks, body, 0)
        pltpu.sync_copy(rows, o_hbm.at[pl.ds(base, rows_per_tile)])

    return k(table, idx)


def _sc_segsum(x, src_idx, dst_idx, s_pad):
    nnz_pad = src_idx.shape[0]
    per_tile = nnz_pad // _NT
    n_chunks = per_tile // _CHUNK
    rows_per_tile = s_pad // _NS
    zrows = 160

    @functools.partial(
        pl.kernel,
        out_type=jax.ShapeDtypeStruct((_NC, s_pad, _D), jnp.float32),
        mesh=_mesh(),
        scratch_types=[
            pltpu.VMEM((_CHUNK,), jnp.int32),
            pltpu.VMEM((_CHUNK,), jnp.int32),
            pltpu.VMEM((_CHUNK,), jnp.int32),
            pltpu.VMEM((_CHUNK,), jnp.int32),
            pltpu.VMEM((_CHUNK, _D), jnp.float32),
            pltpu.VMEM((zrows, _D), jnp.float32),
            pltpu.VMEM_SHARED((s_pad, _D), jnp.float32),
            pltpu.SemaphoreType.DMA,
            pltpu.SemaphoreType.DMA,
            pltpu.SemaphoreType.DMA,
        ],
    )
    def k(x_hbm, src_hbm, dst_hbm, out_hbm, sidx0, sidx1, didx0, didx1, rows,
          zbuf, acc, sem, isem0, isem1):
        cid = lax.axis_index("c")
        sid = lax.axis_index("s")
        wid = sid * _NC + cid
        _zero_vmem(zbuf, zrows, _D)
        base = sid * rows_per_tile
        for t in range(rows_per_tile // zrows):
            pltpu.sync_copy(zbuf, acc.at[pl.ds(base + t * zrows, zrows)])
        plsc.subcore_barrier()

        nbase = wid * per_tile
        sidx = (sidx0, sidx1)
        didx = (didx0, didx1)
        isem = (isem0, isem1)

        def issue_idx(j, b):
            off = nbase + j * _CHUNK
            pltpu.async_copy(src_hbm.at[pl.ds(off, _CHUNK)], sidx[b], isem[b])
            pltpu.async_copy(dst_hbm.at[pl.ds(off, _CHUNK)], didx[b], isem[b])

        def wait_idx(j, b):
            off = nbase + j * _CHUNK
            pltpu.make_async_copy(src_hbm.at[pl.ds(off, _CHUNK)], sidx[b],
                                  isem[b]).wait()
            pltpu.make_async_copy(dst_hbm.at[pl.ds(off, _CHUNK)], didx[b],
                                  isem[b]).wait()

        def step(j, b, prefetch):
            wait_idx(j, b)
            pltpu.async_copy(x_hbm.at[sidx[b]], rows, sem)
            if prefetch:
                issue_idx(j + 1, 1 - b)
            pltpu.make_async_copy(x_hbm.at[sidx[b]], rows, sem).wait()
            pltpu.sync_copy(rows, acc.at[didx[b]], add=True)

        issue_idx(0, 0)

        def body(k2, _):
            for b in range(2):
                step(k2 * 2 + b, b, True)
            return 0
        lax.fori_loop(0, (n_chunks - 1) // 2, body, 0)
        step(n_chunks - 1, (n_chunks - 1) % 2, False)

        plsc.subcore_barrier()
        pltpu.sync_copy(acc.at[pl.ds(base, rows_per_tile)],
                        out_hbm.at[cid, pl.ds(base, rows_per_tile)])

    return k(x, src_idx, dst_idx)


def _tc_linear_relu(x, w, b):
    n, din = x.shape
    dout = w.shape[1]
    block = 1280
    grid = n // block

    def body(x_ref, w_ref, b_ref, o_ref):
        y = jnp.dot(x_ref[...], w_ref[...],
                    preferred_element_type=jnp.float32,
                    precision=lax.Precision.HIGHEST)
        o_ref[...] = jnp.maximum(y + b_ref[...], 0.0)

    return pl.pallas_call(
        body,
        grid=(grid,),
        in_specs=[
            pl.BlockSpec((block, din), lambda i: (i, 0)),
            pl.BlockSpec((din, dout), lambda i: (0, 0)),
            pl.BlockSpec((1, dout), lambda i: (0, 0)),
        ],
        out_specs=pl.BlockSpec((block, dout), lambda i: (i, 0)),
        out_shape=jax.ShapeDtypeStruct((n, dout), jnp.float32),
    )(x, w, b[None, :])


def _tc_tail(ft, yb, Wmu, bmu, Wlv, blv, W3, b3, gamma, beta, Wc, bc, Wd, bd):
    n = ft.shape[0]
    nl = yb.shape[1]
    zdim = Wmu.shape[1]
    block = 1000
    grid = n // block
    dotf = functools.partial(jnp.dot, preferred_element_type=jnp.float32,
                             precision=lax.Precision.HIGHEST)
    g2 = (gamma / jnp.sqrt(1.0 + 1e-5))[None, :]
    Wmu_x, Wmu_y = Wmu[:_D], Wmu[_D:]
    Wlv_x, Wlv_y = Wlv[:_D], Wlv[_D:]
    Wd_z, Wd_y = Wd[:zdim], Wd[zdim:]

    def body(ft_ref, yb_ref, wmux, wmuy, bmu_r, wlvx, wlvy, blv_r, w3, b3_r,
             g2_r, beta_r, wc, bc_r, wdz, wdy, bd_r,
             mu_o, lv_o, lg_o, xr_o):
        ftb = ft_ref[...]
        ybb = yb_ref[...]
        mu = dotf(ftb, wmux[...]) + dotf(ybb, wmuy[...]) + bmu_r[...]
        lv = dotf(ftb, wlvx[...]) + dotf(ybb, wlvy[...]) + blv_r[...]
        h = dotf(mu, w3[...]) + b3_r[...]
        h = jnp.maximum(g2_r[...] * h + beta_r[...], 0.0)
        lg = dotf(h, wc[...]) + bc_r[...]
        xr = dotf(mu, wdz[...]) + dotf(ybb, wdy[...]) + bd_r[...]
        mu_o[...] = mu
        lv_o[...] = lv
        lg_o[...] = lg
        xr_o[...] = xr

    row_spec = lambda d: pl.BlockSpec((block, d), lambda i: (i, 0))
    full = lambda a: pl.BlockSpec(a.shape, lambda i: (0,) * a.ndim)
    args = (ft, yb, Wmu_x, Wmu_y, bmu[None, :], Wlv_x, Wlv_y, blv[None, :],
            W3, b3[None, :], g2, beta[None, :], Wc, bc[None, :],
            Wd_z, Wd_y, bd[None, :])
    in_specs = [row_spec(_D), row_spec(nl)] + [full(a) for a in args[2:]]
    return pl.pallas_call(
        body,
        grid=(grid,),
        in_specs=in_specs,
        out_specs=(row_spec(zdim), row_spec(zdim), row_spec(nl), row_spec(_D)),
        out_shape=(
            jax.ShapeDtypeStruct((n, zdim), jnp.float32),
            jax.ShapeDtypeStruct((n, zdim), jnp.float32),
            jax.ShapeDtypeStruct((n, nl), jnp.float32),
            jax.ShapeDtypeStruct((n, _D), jnp.float32),
        ),
    )(*args)


def kernel(feature, v_idx, e_idx, y_bin, y_target, drug_matrix, new_data_idx,
           W1, b1, W2, b2, Wmu, bmu, Wlv, blv, W3, b3, gamma, beta, Wc, bc,
           Wd, bd):
    n_nodes, d = feature.shape
    nnz = v_idx.shape[0]
    n_he = 5000
    train = n_nodes - y_bin.shape[0]

    unit = _NT * _CHUNK
    nnz_pad = ((nnz + unit - 1) // unit) * unit
    e_pad = ((n_he + 1 + 255) // 256) * 256
    v_pad = ((n_nodes + 1 + 255) // 256) * 256

    v_idx = v_idx.astype(jnp.int32)
    e_idx = e_idx.astype(jnp.int32)
    v_idx_p = jnp.concatenate(
        [v_idx, jnp.full((nnz_pad - nnz,), n_nodes, jnp.int32)])
    e_idx_p = jnp.concatenate(
        [e_idx, jnp.full((nnz_pad - nnz,), n_he, jnp.int32)])
    nidx_p = jnp.concatenate(
        [new_data_idx.astype(jnp.int32),
         jnp.zeros((v_pad - n_nodes,), jnp.int32)])
    feat_in = jnp.pad(feature, ((0, v_pad - n_nodes), (0, 0)))

    e_rcp = _tc_rcp(_sc_count(e_idx_p, e_pad))
    v_rcp = _tc_rcp(_sc_count(v_idx_p, v_pad))
    mol = _sc_gather_rows(drug_matrix, nidx_p)

    x1 = _tc_linear_relu(feat_in, W1, b1)
    ep = _sc_segsum(x1, v_idx_p, e_idx_p, e_pad)
    e_feat = _tc_combine(ep, e_rcp, mode=0)
    vp = _sc_segsum(e_feat, e_idx_p, v_idx_p, v_pad)
    x1m = _tc_combine(vp, v_rcp, mode=1)

    x2 = _tc_linear_relu(x1m, W2, b2)
    ep2 = _sc_segsum(x2, v_idx_p, e_idx_p, e_pad)
    e_feat2 = _tc_combine(ep2, e_rcp, mode=0)
    vp2 = _sc_segsum(e_feat2, e_idx_p, v_idx_p, v_pad)
    feat_full = _tc_combine(vp2, v_rcp, mode=2, mol=mol)

    feat = feat_full[:n_nodes]
    ft = feat_full[train:n_nodes]
    mu, logvar, logits, x_reconst = _tc_tail(
        ft, y_bin, Wmu, bmu, Wlv, blv, W3, b3, gamma, beta, Wc, bc, Wd, bd)
    return (mu, logvar, mu, logits, x_reconst, y_target, feat)


# speedup vs baseline: 1.6627x; 1.1145x over previous
import functools

import jax
import jax.numpy as jnp
from jax import lax
from jax.experimental import pallas as pl
from jax.experimental.pallas import tpu as pltpu
from jax.experimental.pallas import tpu_sc as plsc

_NC = 2
_NS = 16
_NT = _NC * _NS
_L = 16
_CHUNK = 128
_D = 128


def _mesh():
    return plsc.VectorSubcoreMesh(core_axis_name="c", subcore_axis_name="s")


def _zero_vmem(ref, rows, width):
    def body(r, _):
        for k in range(width // _L):
            ref[r, pl.ds(k * _L, _L)] = jnp.zeros((_L,), jnp.float32)
        return 0
    lax.fori_loop(0, rows, body, 0)


def _sc_count(idx_arr, s_pad):
    nnz_pad = idx_arr.shape[0]
    per_tile = nnz_pad // _NT
    n_chunks = per_tile // _CHUNK
    s_rows = s_pad // _NS
    zrows = 160

    @functools.partial(
        pl.kernel,
        out_type=jax.ShapeDtypeStruct((_NC, s_pad, _D), jnp.float32),
        mesh=_mesh(),
        scratch_types=[
            pltpu.VMEM((_CHUNK,), jnp.int32),
            pltpu.VMEM((_CHUNK, _D), jnp.float32),
            pltpu.VMEM((zrows, _D), jnp.float32),
            pltpu.VMEM_SHARED((s_pad, _D), jnp.float32),
        ],
    )
    def k(i_hbm, o_hbm, idx, ones, zbuf, acc):
        cid = lax.axis_index("c")
        sid = lax.axis_index("s")
        wid = sid * _NC + cid

        def fill_ones(r, _):
            for kk in range(_D // _L):
                ones[r, pl.ds(kk * _L, _L)] = jnp.ones((_L,), jnp.float32)
            return 0
        lax.fori_loop(0, _CHUNK, fill_ones, 0)
        _zero_vmem(zbuf, zrows, _D)
        base = sid * s_rows
        for t in range(s_rows // zrows):
            pltpu.sync_copy(zbuf, acc.at[pl.ds(base + t * zrows, zrows)])
        plsc.subcore_barrier()

        nbase = wid * per_tile

        def body(j, _):
            pltpu.sync_copy(i_hbm.at[pl.ds(nbase + j * _CHUNK, _CHUNK)], idx)
            pltpu.sync_copy(ones, acc.at[idx], add=True)
            return 0
        lax.fori_loop(0, n_chunks, body, 0)

        plsc.subcore_barrier()
        pltpu.sync_copy(acc.at[pl.ds(base, s_rows)],
                        o_hbm.at[cid, pl.ds(base, s_rows)])

    return k(idx_arr)


def _tc_rcp(parts):
    s_pad = parts.shape[1]

    def body(a_ref, b_ref, o_ref):
        o_ref[...] = 1.0 / jnp.maximum(a_ref[...] + b_ref[...], 1.0)

    out = pl.pallas_call(
        body,
        out_shape=jax.ShapeDtypeStruct((s_pad, _D), jnp.float32),
    )(parts[0], parts[1])
    return out[:, :1]


def _tc_combine(parts, rcp_col, mode, mol=None):
    s_pad = parts.shape[1]
    block = 1280
    grid = s_pad // block
    row_spec = pl.BlockSpec((block, _D), lambda i: (i, 0))
    col_spec = pl.BlockSpec((block, 1), lambda i: (i, 0))

    def body(*refs):
        if mode == 2:
            a_ref, b_ref, r_ref, m_ref, o_ref = refs
        else:
            a_ref, b_ref, r_ref, o_ref = refs
        y = (a_ref[...] + b_ref[...]) * r_ref[...]
        if mode == 1:
            y = jnp.maximum(y, 0.0)
        if mode == 2:
            y = y + m_ref[...]
        o_ref[...] = y

    args = [parts[0], parts[1], rcp_col]
    in_specs = [row_spec, row_spec, col_spec]
    if mode == 2:
        args.append(mol)
        in_specs.append(row_spec)
    return pl.pallas_call(
        body,
        grid=(grid,),
        in_specs=in_specs,
        out_specs=row_spec,
        out_shape=jax.ShapeDtypeStruct((s_pad, _D), jnp.float32),
    )(*args)


def _sc_gather_rows(table, idx):
    n = idx.shape[0]
    rows_per_tile = n // _NT
    gchunk = 80
    n_chunks = rows_per_tile // gchunk

    @functools.partial(
        pl.kernel,
        out_type=jax.ShapeDtypeStruct((n, _D), jnp.float32),
        mesh=_mesh(),
        scratch_types=[
            pltpu.VMEM((gchunk,), jnp.int32),
            pltpu.VMEM((rows_per_tile, _D), jnp.float32),
            pltpu.SemaphoreType.DMA,
        ],
    )
    def k(t_hbm, i_hbm, o_hbm, gidx, rows, sem):
        cid = lax.axis_index("c")
        sid = lax.axis_index("s")
        wid = sid * _NC + cid
        base = wid * rows_per_tile

        def body(t, _):
            pltpu.sync_copy(i_hbm.at[pl.ds(base + t * gchunk, gchunk)], gidx)
            pltpu.async_copy(t_hbm.at[gidx], rows.at[pl.ds(t * gchunk, gchunk)],
                             sem).wait()
            return 0
        lax.fori_loop(0, n_chunks, body, 0)
        pltpu.sync_copy(rows, o_hbm.at[pl.ds(base, rows_per_tile)])

    return k(table, idx)


def _sc_segsum(x, src_idx, dst_idx, s_pad):
    nnz_pad = src_idx.shape[0]
    per_tile = nnz_pad // _NT
    n_chunks = per_tile // _CHUNK
    rows_per_tile = s_pad // _NS
    zrows = 40

    @functools.partial(
        pl.kernel,
        out_type=jax.ShapeDtypeStruct((_NC, s_pad, _D), jnp.float32),
        mesh=_mesh(),
        scratch_types=[
            pltpu.VMEM((_CHUNK,), jnp.int32),
            pltpu.VMEM((_CHUNK,), jnp.int32),
            pltpu.VMEM((_CHUNK,), jnp.int32),
            pltpu.VMEM((_CHUNK,), jnp.int32),
            pltpu.VMEM((_CHUNK, _D), jnp.float32),
            pltpu.VMEM((_CHUNK, _D), jnp.float32),
            pltpu.VMEM((zrows, _D), jnp.float32),
            pltpu.VMEM_SHARED((s_pad, _D), jnp.float32),
            pltpu.SemaphoreType.DMA,
            pltpu.SemaphoreType.DMA,
            pltpu.SemaphoreType.DMA,
            pltpu.SemaphoreType.DMA,
            pltpu.SemaphoreType.DMA,
        ],
    )
    def k(x_hbm, src_hbm, dst_hbm, out_hbm, sidx0, sidx1, didx0, didx1, rows0,
          rows1, zbuf, acc, g0, g1, isem0, isem1, ssem):
        cid = lax.axis_index("c")
        sid = lax.axis_index("s")
        wid = sid * _NC + cid
        _zero_vmem(zbuf, zrows, _D)
        base = sid * rows_per_tile
        for t in range(rows_per_tile // zrows):
            pltpu.sync_copy(zbuf, acc.at[pl.ds(base + t * zrows, zrows)])
        plsc.subcore_barrier()

        nbase = wid * per_tile
        sidx = (sidx0, sidx1)
        didx = (didx0, didx1)
        isem = (isem0, isem1)

        def issue_idx(j, b):
            off = nbase + j * _CHUNK
            pltpu.async_copy(src_hbm.at[pl.ds(off, _CHUNK)], sidx[b], isem[b])
            pltpu.async_copy(dst_hbm.at[pl.ds(off, _CHUNK)], didx[b], isem[b])

        def wait_idx(j, b):
            off = nbase + j * _CHUNK
            pltpu.make_async_copy(src_hbm.at[pl.ds(off, _CHUNK)], sidx[b],
                                  isem[b]).wait()
            pltpu.make_async_copy(dst_hbm.at[pl.ds(off, _CHUNK)], didx[b],
                                  isem[b]).wait()

        rows = (rows0, rows1)
        gsem = (g0, g1)

        def step(j, b, prefetch, wait_sct):
            wait_idx(j, b)
            if wait_sct:
                pltpu.make_async_copy(rows[b], acc.at[didx[b]], ssem).wait()
            pltpu.async_copy(x_hbm.at[sidx[b]], rows[b], gsem[b])
            if prefetch:
                issue_idx(j + 1, 1 - b)
            pltpu.make_async_copy(x_hbm.at[sidx[b]], rows[b], gsem[b]).wait()
            pltpu.async_copy(rows[b], acc.at[didx[b]], ssem, add=True)

        issue_idx(0, 0)
        step(0, 0, True, False)
        step(1, 1, True, False)

        def body(k2, _):
            for b in range(2):
                step(k2 * 2 + 2 + b, b, True, True)
            return 0
        lax.fori_loop(0, (n_chunks - 3) // 2, body, 0)
        step(n_chunks - 1, (n_chunks - 1) % 2, False, True)
        pltpu.make_async_copy(rows[0], acc.at[didx[0]], ssem).wait()
        pltpu.make_async_copy(rows[1], acc.at[didx[1]], ssem).wait()

        plsc.subcore_barrier()
        pltpu.sync_copy(acc.at[pl.ds(base, rows_per_tile)],
                        out_hbm.at[cid, pl.ds(base, rows_per_tile)])

    return k(x, src_idx, dst_idx)


def _tc_linear_relu(x, w, b):
    n, din = x.shape
    dout = w.shape[1]
    block = 1280
    grid = n // block

    def body(x_ref, w_ref, b_ref, o_ref):
        y = jnp.dot(x_ref[...], w_ref[...],
                    preferred_element_type=jnp.float32,
                    precision=lax.Precision.HIGHEST)
        o_ref[...] = jnp.maximum(y + b_ref[...], 0.0)

    return pl.pallas_call(
        body,
        grid=(grid,),
        in_specs=[
            pl.BlockSpec((block, din), lambda i: (i, 0)),
            pl.BlockSpec((din, dout), lambda i: (0, 0)),
            pl.BlockSpec((1, dout), lambda i: (0, 0)),
        ],
        out_specs=pl.BlockSpec((block, dout), lambda i: (i, 0)),
        out_shape=jax.ShapeDtypeStruct((n, dout), jnp.float32),
    )(x, w, b[None, :])


def _tc_tail(ft, yb, Wmu, bmu, Wlv, blv, W3, b3, gamma, beta, Wc, bc, Wd, bd):
    n = ft.shape[0]
    nl = yb.shape[1]
    zdim = Wmu.shape[1]
    block = 1000
    grid = n // block
    dotf = functools.partial(jnp.dot, preferred_element_type=jnp.float32,
                             precision=lax.Precision.HIGHEST)
    g2 = (gamma / jnp.sqrt(1.0 + 1e-5))[None, :]
    Wmu_x, Wmu_y = Wmu[:_D], Wmu[_D:]
    Wlv_x, Wlv_y = Wlv[:_D], Wlv[_D:]
    Wd_z, Wd_y = Wd[:zdim], Wd[zdim:]

    def body(ft_ref, yb_ref, wmux, wmuy, bmu_r, wlvx, wlvy, blv_r, w3, b3_r,
             g2_r, beta_r, wc, bc_r, wdz, wdy, bd_r,
             mu_o, lv_o, lg_o, xr_o):
        ftb = ft_ref[...]
        ybb = yb_ref[...]
        mu = dotf(ftb, wmux[...]) + dotf(ybb, wmuy[...]) + bmu_r[...]
        lv = dotf(ftb, wlvx[...]) + dotf(ybb, wlvy[...]) + blv_r[...]
        h = dotf(mu, w3[...]) + b3_r[...]
        h = jnp.maximum(g2_r[...] * h + beta_r[...], 0.0)
        lg = dotf(h, wc[...]) + bc_r[...]
        xr = dotf(mu, wdz[...]) + dotf(ybb, wdy[...]) + bd_r[...]
        mu_o[...] = mu
        lv_o[...] = lv
        lg_o[...] = lg
        xr_o[...] = xr

    row_spec = lambda d: pl.BlockSpec((block, d), lambda i: (i, 0))
    full = lambda a: pl.BlockSpec(a.shape, lambda i: (0,) * a.ndim)
    args = (ft, yb, Wmu_x, Wmu_y, bmu[None, :], Wlv_x, Wlv_y, blv[None, :],
            W3, b3[None, :], g2, beta[None, :], Wc, bc[None, :],
            Wd_z, Wd_y, bd[None, :])
    in_specs = [row_spec(_D), row_spec(nl)] + [full(a) for a in args[2:]]
    return pl.pallas_call(
        body,
        grid=(grid,),
        in_specs=in_specs,
        out_specs=(row_spec(zdim), row_spec(zdim), row_spec(nl), row_spec(_D)),
        out_shape=(
            jax.ShapeDtypeStruct((n, zdim), jnp.float32),
            jax.ShapeDtypeStruct((n, zdim), jnp.float32),
            jax.ShapeDtypeStruct((n, nl), jnp.float32),
            jax.ShapeDtypeStruct((n, _D), jnp.float32),
        ),
    )(*args)


def kernel(feature, v_idx, e_idx, y_bin, y_target, drug_matrix, new_data_idx,
           W1, b1, W2, b2, Wmu, bmu, Wlv, blv, W3, b3, gamma, beta, Wc, bc,
           Wd, bd):
    n_nodes, d = feature.shape
    nnz = v_idx.shape[0]
    n_he = 5000
    train = n_nodes - y_bin.shape[0]

    unit = _NT * _CHUNK
    nnz_pad = ((nnz + unit - 1) // unit) * unit
    e_pad = ((n_he + 1 + 255) // 256) * 256
    v_pad = ((n_nodes + 1 + 255) // 256) * 256

    v_idx = v_idx.astype(jnp.int32)
    e_idx = e_idx.astype(jnp.int32)
    v_idx_p = jnp.concatenate(
        [v_idx, jnp.full((nnz_pad - nnz,), n_nodes, jnp.int32)])
    e_idx_p = jnp.concatenate(
        [e_idx, jnp.full((nnz_pad - nnz,), n_he, jnp.int32)])
    nidx_p = jnp.concatenate(
        [new_data_idx.astype(jnp.int32),
         jnp.zeros((v_pad - n_nodes,), jnp.int32)])
    feat_in = jnp.pad(feature, ((0, v_pad - n_nodes), (0, 0)))

    e_rcp = _tc_rcp(_sc_count(e_idx_p, e_pad))
    v_rcp = _tc_rcp(_sc_count(v_idx_p, v_pad))
    mol = _sc_gather_rows(drug_matrix, nidx_p)

    x1 = _tc_linear_relu(feat_in, W1, b1)
    ep = _sc_segsum(x1, v_idx_p, e_idx_p, e_pad)
    e_feat = _tc_combine(ep, e_rcp, mode=0)
    vp = _sc_segsum(e_feat, e_idx_p, v_idx_p, v_pad)
    x1m = _tc_combine(vp, v_rcp, mode=1)

    x2 = _tc_linear_relu(x1m, W2, b2)
    ep2 = _sc_segsum(x2, v_idx_p, e_idx_p, e_pad)
    e_feat2 = _tc_combine(ep2, e_rcp, mode=0)
    vp2 = _sc_segsum(e_feat2, e_idx_p, v_idx_p, v_pad)
    feat_full = _tc_combine(vp2, v_rcp, mode=2, mol=mol)

    feat = feat_full[:n_nodes]
    ft = feat_full[train:n_nodes]
    mu, logvar, logits, x_reconst = _tc_tail(
        ft, y_bin, Wmu, bmu, Wlv, blv, W3, b3, gamma, beta, Wc, bc, Wd, bd)
    return (mu, logvar, mu, logits, x_reconst, y_target, feat)


# speedup vs baseline: 1.7383x; 1.0455x over previous
import functools

import jax
import jax.numpy as jnp
from jax import lax
from jax.experimental import pallas as pl
from jax.experimental.pallas import tpu as pltpu
from jax.experimental.pallas import tpu_sc as plsc

_NC = 2
_NS = 16
_NT = _NC * _NS
_L = 16
_CHUNK = 128
_D = 128


def _mesh():
    return plsc.VectorSubcoreMesh(core_axis_name="c", subcore_axis_name="s")


def _zero_vmem(ref, rows, width):
    def body(r, _):
        for k in range(width // _L):
            ref[r, pl.ds(k * _L, _L)] = jnp.zeros((_L,), jnp.float32)
        return 0
    lax.fori_loop(0, rows, body, 0)


def _sc_count(idx_arr, s_pad):
    nnz_pad = idx_arr.shape[0]
    per_tile = nnz_pad // _NT
    n_chunks = per_tile // _CHUNK
    s_rows = s_pad // _NS
    zrows = 160

    @functools.partial(
        pl.kernel,
        out_type=jax.ShapeDtypeStruct((_NC, s_pad, _D), jnp.float32),
        mesh=_mesh(),
        scratch_types=[
            pltpu.VMEM((_CHUNK,), jnp.int32),
            pltpu.VMEM((_CHUNK,), jnp.int32),
            pltpu.VMEM((_CHUNK, _D), jnp.float32),
            pltpu.VMEM((zrows, _D), jnp.float32),
            pltpu.VMEM_SHARED((s_pad, _D), jnp.float32),
            pltpu.SemaphoreType.DMA,
            pltpu.SemaphoreType.DMA,
            pltpu.SemaphoreType.DMA,
        ],
    )
    def k(i_hbm, o_hbm, idx0, idx1, ones, zbuf, acc, i0, i1, ssem):
        cid = lax.axis_index("c")
        sid = lax.axis_index("s")
        wid = sid * _NC + cid

        def fill_ones(r, _):
            for kk in range(_D // _L):
                ones[r, pl.ds(kk * _L, _L)] = jnp.ones((_L,), jnp.float32)
            return 0
        lax.fori_loop(0, _CHUNK, fill_ones, 0)
        _zero_vmem(zbuf, zrows, _D)
        base = sid * s_rows
        for t in range(s_rows // zrows):
            pltpu.sync_copy(zbuf, acc.at[pl.ds(base + t * zrows, zrows)])
        plsc.subcore_barrier()

        nbase = wid * per_tile
        idx = (idx0, idx1)
        isem = (i0, i1)

        def issue_idx(j, b):
            pltpu.async_copy(i_hbm.at[pl.ds(nbase + j * _CHUNK, _CHUNK)],
                             idx[b], isem[b])

        def step(j, b, prefetch, wait_sct):
            pltpu.make_async_copy(
                i_hbm.at[pl.ds(nbase + j * _CHUNK, _CHUNK)], idx[b],
                isem[b]).wait()
            if wait_sct:
                pltpu.make_async_copy(ones, acc.at[idx[1 - b]], ssem).wait()
            if prefetch:
                issue_idx(j + 1, 1 - b)
            pltpu.async_copy(ones, acc.at[idx[b]], ssem, add=True)

        issue_idx(0, 0)
        step(0, 0, True, False)

        def body(k2, _):
            for bb in range(2):
                j = 1 + k2 * 2 + bb
                step(j, (1 + bb) % 2, True, True)
            return 0
        lax.fori_loop(0, (n_chunks - 3) // 2, body, 0)
        step(n_chunks - 2, (n_chunks - 2) % 2, True, True)
        step(n_chunks - 1, (n_chunks - 1) % 2, False, True)
        pltpu.make_async_copy(ones, acc.at[idx[(n_chunks - 1) % 2]],
                              ssem).wait()

        plsc.subcore_barrier()
        pltpu.sync_copy(acc.at[pl.ds(base, s_rows)],
                        o_hbm.at[cid, pl.ds(base, s_rows)])

    return k(idx_arr)


def _tc_rcp(parts):
    s_pad = parts.shape[1]

    def body(a_ref, b_ref, o_ref):
        o_ref[...] = 1.0 / jnp.maximum(a_ref[...] + b_ref[...], 1.0)

    out = pl.pallas_call(
        body,
        out_shape=jax.ShapeDtypeStruct((s_pad, _D), jnp.float32),
    )(parts[0], parts[1])
    return out[:, :1]


def _tc_combine(parts, rcp_col, mode, mol=None):
    s_pad = parts.shape[1]
    block = 1280
    grid = s_pad // block
    row_spec = pl.BlockSpec((block, _D), lambda i: (i, 0))
    col_spec = pl.BlockSpec((block, 1), lambda i: (i, 0))

    def body(*refs):
        if mode == 2:
            a_ref, b_ref, r_ref, m_ref, o_ref = refs
        else:
            a_ref, b_ref, r_ref, o_ref = refs
        y = (a_ref[...] + b_ref[...]) * r_ref[...]
        if mode == 1:
            y = jnp.maximum(y, 0.0)
        if mode == 2:
            y = y + m_ref[...]
        o_ref[...] = y

    args = [parts[0], parts[1], rcp_col]
    in_specs = [row_spec, row_spec, col_spec]
    if mode == 2:
        args.append(mol)
        in_specs.append(row_spec)
    return pl.pallas_call(
        body,
        grid=(grid,),
        in_specs=in_specs,
        out_specs=row_spec,
        out_shape=jax.ShapeDtypeStruct((s_pad, _D), jnp.float32),
    )(*args)


def _sc_gather_rows(table, idx):
    n = idx.shape[0]
    rows_per_tile = n // _NT
    gchunk = 80
    n_chunks = rows_per_tile // gchunk

    @functools.partial(
        pl.kernel,
        out_type=jax.ShapeDtypeStruct((n, _D), jnp.float32),
        mesh=_mesh(),
        scratch_types=[
            pltpu.VMEM((gchunk,), jnp.int32),
            pltpu.VMEM((rows_per_tile, _D), jnp.float32),
            pltpu.SemaphoreType.DMA,
        ],
    )
    def k(t_hbm, i_hbm, o_hbm, gidx, rows, sem):
        cid = lax.axis_index("c")
        sid = lax.axis_index("s")
        wid = sid * _NC + cid
        base = wid * rows_per_tile

        def body(t, _):
            pltpu.sync_copy(i_hbm.at[pl.ds(base + t * gchunk, gchunk)], gidx)
            pltpu.async_copy(t_hbm.at[gidx], rows.at[pl.ds(t * gchunk, gchunk)],
                             sem).wait()
            return 0
        lax.fori_loop(0, n_chunks, body, 0)
        pltpu.sync_copy(rows, o_hbm.at[pl.ds(base, rows_per_tile)])

    return k(table, idx)


def _sc_segsum(x, src_idx, dst_idx, s_pad):
    nnz_pad = src_idx.shape[0]
    per_tile = nnz_pad // _NT
    n_chunks = per_tile // _CHUNK
    rows_per_tile = s_pad // _NS
    zrows = 40

    @functools.partial(
        pl.kernel,
        out_type=jax.ShapeDtypeStruct((_NC, s_pad, _D), jnp.float32),
        mesh=_mesh(),
        scratch_types=[
            pltpu.VMEM((_CHUNK,), jnp.int32),
            pltpu.VMEM((_CHUNK,), jnp.int32),
            pltpu.VMEM((_CHUNK,), jnp.int32),
            pltpu.VMEM((_CHUNK,), jnp.int32),
            pltpu.VMEM((_CHUNK, _D), jnp.float32),
            pltpu.VMEM((_CHUNK, _D), jnp.float32),
            pltpu.VMEM((zrows, _D), jnp.float32),
            pltpu.VMEM_SHARED((s_pad, _D), jnp.float32),
            pltpu.SemaphoreType.DMA,
            pltpu.SemaphoreType.DMA,
            pltpu.SemaphoreType.DMA,
            pltpu.SemaphoreType.DMA,
            pltpu.SemaphoreType.DMA,
        ],
    )
    def k(x_hbm, src_hbm, dst_hbm, out_hbm, sidx0, sidx1, didx0, didx1, rows0,
          rows1, zbuf, acc, g0, g1, isem0, isem1, ssem):
        cid = lax.axis_index("c")
        sid = lax.axis_index("s")
        wid = sid * _NC + cid
        _zero_vmem(zbuf, zrows, _D)
        base = sid * rows_per_tile
        for t in range(rows_per_tile // zrows):
            pltpu.sync_copy(zbuf, acc.at[pl.ds(base + t * zrows, zrows)])
        plsc.subcore_barrier()

        nbase = wid * per_tile
        sidx = (sidx0, sidx1)
        didx = (didx0, didx1)
        isem = (isem0, isem1)

        def issue_idx(j, b):
            off = nbase + j * _CHUNK
            pltpu.async_copy(src_hbm.at[pl.ds(off, _CHUNK)], sidx[b], isem[b])
            pltpu.async_copy(dst_hbm.at[pl.ds(off, _CHUNK)], didx[b], isem[b])

        def wait_idx(j, b):
            off = nbase + j * _CHUNK
            pltpu.make_async_copy(src_hbm.at[pl.ds(off, _CHUNK)], sidx[b],
                                  isem[b]).wait()
            pltpu.make_async_copy(dst_hbm.at[pl.ds(off, _CHUNK)], didx[b],
                                  isem[b]).wait()

        rows = (rows0, rows1)
        gsem = (g0, g1)

        def step(j, b, prefetch, wait_sct):
            wait_idx(j, b)
            if wait_sct:
                pltpu.make_async_copy(rows[b], acc.at[didx[b]], ssem).wait()
            pltpu.async_copy(x_hbm.at[sidx[b]], rows[b], gsem[b])
            if prefetch:
                issue_idx(j + 1, 1 - b)
            pltpu.make_async_copy(x_hbm.at[sidx[b]], rows[b], gsem[b]).wait()
            pltpu.async_copy(rows[b], acc.at[didx[b]], ssem, add=True)

        issue_idx(0, 0)
        step(0, 0, True, False)
        step(1, 1, True, False)

        def body(k2, _):
            for b in range(2):
                step(k2 * 2 + 2 + b, b, True, True)
            return 0
        lax.fori_loop(0, (n_chunks - 3) // 2, body, 0)
        step(n_chunks - 1, (n_chunks - 1) % 2, False, True)
        pltpu.make_async_copy(rows[0], acc.at[didx[0]], ssem).wait()
        pltpu.make_async_copy(rows[1], acc.at[didx[1]], ssem).wait()

        plsc.subcore_barrier()
        pltpu.sync_copy(acc.at[pl.ds(base, rows_per_tile)],
                        out_hbm.at[cid, pl.ds(base, rows_per_tile)])

    return k(x, src_idx, dst_idx)


def _tc_linear_relu(x, w, b):
    n, din = x.shape
    dout = w.shape[1]
    block = 1280
    grid = n // block

    def body(x_ref, w_ref, b_ref, o_ref):
        y = jnp.dot(x_ref[...], w_ref[...],
                    preferred_element_type=jnp.float32,
                    precision=lax.Precision.HIGHEST)
        o_ref[...] = jnp.maximum(y + b_ref[...], 0.0)

    return pl.pallas_call(
        body,
        grid=(grid,),
        in_specs=[
            pl.BlockSpec((block, din), lambda i: (i, 0)),
            pl.BlockSpec((din, dout), lambda i: (0, 0)),
            pl.BlockSpec((1, dout), lambda i: (0, 0)),
        ],
        out_specs=pl.BlockSpec((block, dout), lambda i: (i, 0)),
        out_shape=jax.ShapeDtypeStruct((n, dout), jnp.float32),
    )(x, w, b[None, :])


def _tc_tail(ft, yb, Wmu, bmu, Wlv, blv, W3, b3, gamma, beta, Wc, bc, Wd, bd):
    n = ft.shape[0]
    nl = yb.shape[1]
    zdim = Wmu.shape[1]
    block = 1000
    grid = n // block
    dotf = functools.partial(jnp.dot, preferred_element_type=jnp.float32,
                             precision=lax.Precision.HIGHEST)
    g2 = (gamma / jnp.sqrt(1.0 + 1e-5))[None, :]
    Wmu_x, Wmu_y = Wmu[:_D], Wmu[_D:]
    Wlv_x, Wlv_y = Wlv[:_D], Wlv[_D:]
    Wd_z, Wd_y = Wd[:zdim], Wd[zdim:]

    def body(ft_ref, yb_ref, wmux, wmuy, bmu_r, wlvx, wlvy, blv_r, w3, b3_r,
             g2_r, beta_r, wc, bc_r, wdz, wdy, bd_r,
             mu_o, lv_o, lg_o, xr_o):
        ftb = ft_ref[...]
        ybb = yb_ref[...]
        mu = dotf(ftb, wmux[...]) + dotf(ybb, wmuy[...]) + bmu_r[...]
        lv = dotf(ftb, wlvx[...]) + dotf(ybb, wlvy[...]) + blv_r[...]
        h = dotf(mu, w3[...]) + b3_r[...]
        h = jnp.maximum(g2_r[...] * h + beta_r[...], 0.0)
        lg = dotf(h, wc[...]) + bc_r[...]
        xr = dotf(mu, wdz[...]) + dotf(ybb, wdy[...]) + bd_r[...]
        mu_o[...] = mu
        lv_o[...] = lv
        lg_o[...] = lg
        xr_o[...] = xr

    row_spec = lambda d: pl.BlockSpec((block, d), lambda i: (i, 0))
    full = lambda a: pl.BlockSpec(a.shape, lambda i: (0,) * a.ndim)
    args = (ft, yb, Wmu_x, Wmu_y, bmu[None, :], Wlv_x, Wlv_y, blv[None, :],
            W3, b3[None, :], g2, beta[None, :], Wc, bc[None, :],
            Wd_z, Wd_y, bd[None, :])
    in_specs = [row_spec(_D), row_spec(nl)] + [full(a) for a in args[2:]]
    return pl.pallas_call(
        body,
        grid=(grid,),
        in_specs=in_specs,
        out_specs=(row_spec(zdim), row_spec(zdim), row_spec(nl), row_spec(_D)),
        out_shape=(
            jax.ShapeDtypeStruct((n, zdim), jnp.float32),
            jax.ShapeDtypeStruct((n, zdim), jnp.float32),
            jax.ShapeDtypeStruct((n, nl), jnp.float32),
            jax.ShapeDtypeStruct((n, _D), jnp.float32),
        ),
    )(*args)


def kernel(feature, v_idx, e_idx, y_bin, y_target, drug_matrix, new_data_idx,
           W1, b1, W2, b2, Wmu, bmu, Wlv, blv, W3, b3, gamma, beta, Wc, bc,
           Wd, bd):
    n_nodes, d = feature.shape
    nnz = v_idx.shape[0]
    n_he = 5000
    train = n_nodes - y_bin.shape[0]

    unit = _NT * _CHUNK
    nnz_pad = ((nnz + unit - 1) // unit) * unit
    e_pad = ((n_he + 1 + 255) // 256) * 256
    v_pad = ((n_nodes + 1 + 255) // 256) * 256

    v_idx = v_idx.astype(jnp.int32)
    e_idx = e_idx.astype(jnp.int32)
    v_idx_p = jnp.concatenate(
        [v_idx, jnp.full((nnz_pad - nnz,), n_nodes, jnp.int32)])
    e_idx_p = jnp.concatenate(
        [e_idx, jnp.full((nnz_pad - nnz,), n_he, jnp.int32)])
    nidx_p = jnp.concatenate(
        [new_data_idx.astype(jnp.int32),
         jnp.zeros((v_pad - n_nodes,), jnp.int32)])
    feat_in = jnp.pad(feature, ((0, v_pad - n_nodes), (0, 0)))

    e_rcp = _tc_rcp(_sc_count(e_idx_p, e_pad))
    v_rcp = _tc_rcp(_sc_count(v_idx_p, v_pad))
    mol = _sc_gather_rows(drug_matrix, nidx_p)

    x1 = _tc_linear_relu(feat_in, W1, b1)
    ep = _sc_segsum(x1, v_idx_p, e_idx_p, e_pad)
    e_feat = _tc_combine(ep, e_rcp, mode=0)
    vp = _sc_segsum(e_feat, e_idx_p, v_idx_p, v_pad)
    x1m = _tc_combine(vp, v_rcp, mode=1)

    x2 = _tc_linear_relu(x1m, W2, b2)
    ep2 = _sc_segsum(x2, v_idx_p, e_idx_p, e_pad)
    e_feat2 = _tc_combine(ep2, e_rcp, mode=0)
    vp2 = _sc_segsum(e_feat2, e_idx_p, v_idx_p, v_pad)
    feat_full = _tc_combine(vp2, v_rcp, mode=2, mol=mol)

    feat = feat_full[:n_nodes]
    ft = feat_full[train:n_nodes]
    mu, logvar, logits, x_reconst = _tc_tail(
        ft, y_bin, Wmu, bmu, Wlv, blv, W3, b3, gamma, beta, Wc, bc, Wd, bd)
    return (mu, logvar, mu, logits, x_reconst, y_target, feat)
